# baseline (speedup 1.0000x reference)
import functools

import jax
import jax.numpy as jnp
from jax import lax
from jax.experimental import pallas as pl
from jax.experimental.pallas import tpu as pltpu
from jax.experimental.pallas import tpu_sc as plsc

_SIZE = 1_000_000
_DIM = 64
_BATCH = 16384
_NW = 32
_C = 128
_NWIN = _SIZE // _C
_TAIL_START = _NWIN * _C
_TAIL = _SIZE - _TAIL_START
_CAP = 704
_NP = _NWIN + 2
_NP_PAD = ((_NP + 352) // 32) * 32
_IDX_PAD = _BATCH + 32


_mesh = plsc.VectorSubcoreMesh(core_axis_name="c", subcore_axis_name="s")


@functools.partial(
    pl.kernel,
    mesh=_mesh,
    out_type=jax.ShapeDtypeStruct((_DIM, _SIZE), jnp.float32),
    scratch_types=[
        pltpu.VMEM((320,), jnp.int32),
        pltpu.VMEM((_CAP + 32,), jnp.int32),
        pltpu.VMEM((_CAP + 32,), jnp.int32),
        pltpu.VMEM((_CAP, _DIM), jnp.float32),
        pltpu.VMEM((2, _DIM, _C), jnp.float32),
        pltpu.VMEM((_DIM, _TAIL), jnp.float32),
        pltpu.SemaphoreType.DMA,
        pltpu.SemaphoreType.DMA,
        pltpu.SemaphoreType.DMA((2,)),
        pltpu.SemaphoreType.DMA((2,)),
        pltpu.SemaphoreType.DMA,
    ],
)
def _sc_fused(bank_t, si_hbm, ord_hbm, p_hbm, data_rm, out_t,
              p_v, si_v, ord_v, stage, buf, tailbuf,
              pre_sem, g_sem, in_sems, out_sems, t_sem):
    wid = lax.axis_index("s") * 2 + lax.axis_index("c")
    lo = (wid * _NWIN) // _NW
    hi = ((wid + 1) * _NWIN) // _NW
    is_last = wid == _NW - 1

    lop8 = pl.multiple_of((lo // 8) * 8, 8)
    pltpu.async_copy(p_hbm.at[pl.ds(lop8, 320)], p_v, pre_sem).wait()

    def _pread(i):
        return p_v[pl.ds(i - lop8, 16)][0]

    a = _pread(lo)
    b = _pread(jnp.where(is_last, _NWIN + 1, hi))
    n = jnp.minimum(b - a, _CAP)
    a8 = pl.multiple_of((a // 8) * 8, 8)
    off = a - a8
    pltpu.async_copy(si_hbm.at[pl.ds(a8, _CAP + 32)], si_v, pre_sem).wait()
    pltpu.async_copy(ord_hbm.at[pl.ds(a8, _CAP + 32)], ord_v, pre_sem).wait()

    def fire_stage(t, carry):
        s = ord_v[pl.ds(off + t, 16)][0]
        pltpu.async_copy(data_rm.at[s], stage.at[t], g_sem)
        return carry

    lax.fori_loop(0, n, fire_stage, 0)

    def drain_stage(t, carry):
        pltpu.make_async_copy(data_rm.at[0], stage.at[0], g_sem).wait()
        return carry

    lax.fori_loop(0, n, drain_stage, 0)

    iota16 = lax.iota(jnp.int32, 16)

    def make_apply(target):
        def apply_updates(w, wstart):
            pv = p_v[pl.ds(w - lop8, 16)]
            p0, p1 = pv[0], pv[1]

            def one(p, carry):
                c = si_v[pl.ds(off + (p - a), 16)][0] - wstart
                j = p - a
                seg = pl.multiple_of((c // 16) * 16, 16)
                mask = iota16 == c - seg
                for g in range(4):
                    xvec = stage[j, pl.ds(g * 16, 16)]
                    for l in range(16):
                        r = g * 16 + l
                        xv = jnp.broadcast_to(xvec[l], (16,))
                        cur = target(r, seg)
                        target(r, seg, jnp.where(mask, xv, cur))
                return carry

            lax.fori_loop(p0, p1, one, 0)

        return apply_updates

    def win_start(w):
        return pl.multiple_of(w * _C, 128)

    def issue_in(w, slot):
        pltpu.async_copy(
            bank_t.at[:, pl.ds(win_start(w), _C)], buf.at[slot],
            in_sems.at[slot],
        )

    def wait_in(slot):
        pltpu.make_async_copy(
            bank_t.at[:, pl.ds(0, _C)], buf.at[slot], in_sems.at[slot]
        ).wait()

    def issue_out(w, slot):
        pltpu.async_copy(
            buf.at[slot], out_t.at[:, pl.ds(win_start(w), _C)],
            out_sems.at[slot],
        )

    def wait_out(slot):
        pltpu.make_async_copy(
            buf.at[slot], out_t.at[:, pl.ds(0, _C)], out_sems.at[slot]
        ).wait()

    nw = hi - lo

    @pl.when(nw > 0)
    def _():
        issue_in(lo, 0)

        def body(i, carry):
            for slot in range(2):
                w = lo + 2 * i + slot
                nslot = 1 - slot

                def bslot_rw(r, seg, val=None):
                    if val is None:
                        return buf[slot, r, pl.ds(seg, 16)]
                    buf[slot, r, pl.ds(seg, 16)] = val

                @pl.when(w < hi)
                def _():
                    @pl.when(w + 1 < hi)
                    def _():
                        @pl.when(w > lo)
                        def _():
                            wait_out(nslot)

                        issue_in(w + 1, nslot)

                    wait_in(slot)
                    make_apply(bslot_rw)(w, w * _C)
                    issue_out(w, slot)

            return carry

        lax.fori_loop(0, (nw + 1) // 2, body, 0)
        wait_out(0)

        @pl.when(nw > 1)
        def _():
            wait_out(1)

    @pl.when(is_last)
    def _():
        pltpu.async_copy(
            bank_t.at[:, pl.ds(_TAIL_START, _TAIL)], tailbuf, t_sem
        ).wait()

        def tail_rw(r, seg, val=None):
            if val is None:
                return tailbuf[r, pl.ds(seg, 16)]
            tailbuf[r, pl.ds(seg, 16)] = val

        make_apply(tail_rw)(_NWIN, _TAIL_START)
        pltpu.async_copy(
            tailbuf, out_t.at[:, pl.ds(_TAIL_START, _TAIL)], t_sem
        ).wait()


def kernel(bank, indices, data_memory):
    order = jnp.argsort(indices, stable=True).astype(jnp.int32)
    si = jnp.take(indices, order)
    wstarts = jnp.arange(_NWIN + 1, dtype=jnp.int32) * _C
    p = jnp.searchsorted(si, wstarts, side="left").astype(jnp.int32)
    p = jnp.concatenate([p, jnp.full((1,), _BATCH, jnp.int32)])
    p = jnp.pad(p, (0, _NP_PAD - _NP), constant_values=_BATCH)
    si_p = jnp.pad(si, (0, _IDX_PAD - _BATCH), constant_values=_SIZE)
    ord_p = jnp.pad(order, (0, _IDX_PAD - _BATCH))

    out_t = _sc_fused(bank.T, si_p, ord_p, p, data_memory)
    return out_t.T


# speedup vs baseline: 1.6282x; 1.6282x over previous
import functools

import jax
import jax.numpy as jnp
from jax import lax
from jax.experimental import pallas as pl
from jax.experimental.pallas import tpu as pltpu
from jax.experimental.pallas import tpu_sc as plsc

_SIZE = 1_000_000
_DIM = 64
_BATCH = 16384
_NW = 32
_C = 256
_NWIN = _SIZE // _C
_TAIL_START = _NWIN * _C
_TAIL = _SIZE - _TAIL_START
_CAP = 640
_NP = _NWIN + 2
_NP_PAD = ((_NP + 352) // 32) * 32
_IDX_PAD = _BATCH + 32


_mesh = plsc.VectorSubcoreMesh(core_axis_name="c", subcore_axis_name="s")


@functools.partial(
    pl.kernel,
    mesh=_mesh,
    out_type=jax.ShapeDtypeStruct((_DIM, _SIZE), jnp.float32),
    scratch_types=[
        pltpu.VMEM((192,), jnp.int32),
        pltpu.VMEM((_CAP + 32,), jnp.int32),
        pltpu.VMEM((_CAP + 32,), jnp.int32),
        pltpu.VMEM((_CAP, _DIM), jnp.float32),
        pltpu.VMEM((2, _DIM, _C), jnp.float32),
        pltpu.VMEM((_DIM, _TAIL), jnp.float32),
        pltpu.SemaphoreType.DMA,
        pltpu.SemaphoreType.DMA,
        pltpu.SemaphoreType.DMA((2,)),
        pltpu.SemaphoreType.DMA((2,)),
        pltpu.SemaphoreType.DMA,
    ],
)
def _sc_fused(bank_t, si_hbm, ord_hbm, p_hbm, data_rm, out_t,
              p_v, si_v, ord_v, stage, buf, tailbuf,
              pre_sem, g_sem, in_sems, out_sems, t_sem):
    wid = lax.axis_index("s") * 2 + lax.axis_index("c")
    lo = (wid * _NWIN) // _NW
    hi = ((wid + 1) * _NWIN) // _NW
    is_last = wid == _NW - 1

    lop8 = pl.multiple_of((lo // 8) * 8, 8)
    pltpu.async_copy(p_hbm.at[pl.ds(lop8, 192)], p_v, pre_sem).wait()

    def _pread(i):
        return p_v[pl.ds(i - lop8, 16)][0]

    a = _pread(lo)
    b = _pread(jnp.where(is_last, _NWIN + 1, hi))
    n = jnp.minimum(b - a, _CAP)
    a8 = pl.multiple_of((a // 8) * 8, 8)
    off = a - a8
    pltpu.async_copy(si_hbm.at[pl.ds(a8, _CAP + 32)], si_v, pre_sem).wait()
    pltpu.async_copy(ord_hbm.at[pl.ds(a8, _CAP + 32)], ord_v, pre_sem).wait()

    def fire_stage(t, carry):
        s = ord_v[pl.ds(off + t, 16)][0]
        pltpu.async_copy(data_rm.at[s], stage.at[t], g_sem)
        return carry

    lax.fori_loop(0, n, fire_stage, 0)

    def drain_stage(t, carry):
        pltpu.make_async_copy(data_rm.at[0], stage.at[0], g_sem).wait()
        return carry

    lax.fori_loop(0, n, drain_stage, 0)

    iota16 = lax.iota(jnp.int32, 16)

    def make_apply(target):
        def apply_updates(w, wstart):
            pv = p_v[pl.ds(w - lop8, 16)]
            p0, p1 = pv[0], pv[1]

            def one(p, carry):
                c = si_v[pl.ds(off + (p - a), 16)][0] - wstart
                j = p - a
                seg = pl.multiple_of((c // 16) * 16, 16)
                mask = iota16 == c - seg
                for g in range(4):
                    xvec = stage[j, pl.ds(g * 16, 16)]
                    for l in range(16):
                        r = g * 16 + l
                        xv = jnp.broadcast_to(xvec[l], (16,))
                        cur = target(r, seg)
                        target(r, seg, jnp.where(mask, xv, cur))
                return carry

            lax.fori_loop(p0, p1, one, 0)

        return apply_updates

    def win_start(w):
        return pl.multiple_of(w * _C, 128)

    def issue_in(w, slot):
        pltpu.async_copy(
            bank_t.at[:, pl.ds(win_start(w), _C)], buf.at[slot],
            in_sems.at[slot],
        )

    def wait_in(slot):
        pltpu.make_async_copy(
            bank_t.at[:, pl.ds(0, _C)], buf.at[slot], in_sems.at[slot]
        ).wait()

    def issue_out(w, slot):
        pltpu.async_copy(
            buf.at[slot], out_t.at[:, pl.ds(win_start(w), _C)],
            out_sems.at[slot],
        )

    def wait_out(slot):
        pltpu.make_async_copy(
            buf.at[slot], out_t.at[:, pl.ds(0, _C)], out_sems.at[slot]
        ).wait()

    nw = hi - lo

    @pl.when(nw > 0)
    def _():
        issue_in(lo, 0)

        def body(i, carry):
            for slot in range(2):
                w = lo + 2 * i + slot
                nslot = 1 - slot

                def bslot_rw(r, seg, val=None):
                    if val is None:
                        return buf[slot, r, pl.ds(seg, 16)]
                    buf[slot, r, pl.ds(seg, 16)] = val

                @pl.when(w < hi)
                def _():
                    @pl.when(w + 1 < hi)
                    def _():
                        @pl.when(w > lo)
                        def _():
                            wait_out(nslot)

                        issue_in(w + 1, nslot)

                    wait_in(slot)
                    make_apply(bslot_rw)(w, w * _C)
                    issue_out(w, slot)

            return carry

        lax.fori_loop(0, (nw + 1) // 2, body, 0)
        wait_out(0)

        @pl.when(nw > 1)
        def _():
            wait_out(1)

    @pl.when(is_last)
    def _():
        pltpu.async_copy(
            bank_t.at[:, pl.ds(_TAIL_START, _TAIL)], tailbuf, t_sem
        ).wait()

        def tail_rw(r, seg, val=None):
            if val is None:
                return tailbuf[r, pl.ds(seg, 16)]
            tailbuf[r, pl.ds(seg, 16)] = val

        make_apply(tail_rw)(_NWIN, _TAIL_START)
        pltpu.async_copy(
            tailbuf, out_t.at[:, pl.ds(_TAIL_START, _TAIL)], t_sem
        ).wait()


def kernel(bank, indices, data_memory):
    order = jnp.argsort(indices, stable=True).astype(jnp.int32)
    si = jnp.take(indices, order)
    wstarts = jnp.arange(_NWIN + 1, dtype=jnp.int32) * _C
    p = jnp.searchsorted(si, wstarts, side="left").astype(jnp.int32)
    p = jnp.concatenate([p, jnp.full((1,), _BATCH, jnp.int32)])
    p = jnp.pad(p, (0, _NP_PAD - _NP), constant_values=_BATCH)
    si_p = jnp.pad(si, (0, _IDX_PAD - _BATCH), constant_values=_SIZE)
    ord_p = jnp.pad(order, (0, _IDX_PAD - _BATCH))

    out_t = _sc_fused(bank.T, si_p, ord_p, p, data_memory)
    return out_t.T


# speedup vs baseline: 1.6297x; 1.0009x over previous
import functools

import jax
import jax.numpy as jnp
from jax import lax
from jax.experimental import pallas as pl
from jax.experimental.pallas import tpu as pltpu
from jax.experimental.pallas import tpu_sc as plsc

_SIZE = 1_000_000
_DIM = 64
_BATCH = 16384
_NW = 32
_C = 256
_NWIN = _SIZE // _C
_TAIL_START = _NWIN * _C
_TAIL = _SIZE - _TAIL_START
_CAP = 640
_NP = _NWIN + 2
_NP_PAD = ((_NP + 352) // 32) * 32
_IDX_PAD = _BATCH + 32


_mesh = plsc.VectorSubcoreMesh(core_axis_name="c", subcore_axis_name="s")


@functools.partial(
    pl.kernel,
    mesh=_mesh,
    out_type=jax.ShapeDtypeStruct((_DIM, _SIZE), jnp.float32),
    scratch_types=[
        pltpu.VMEM((192,), jnp.int32),
        pltpu.VMEM((_CAP + 32,), jnp.int32),
        pltpu.VMEM((_CAP + 32,), jnp.int32),
        pltpu.VMEM((_CAP, _DIM), jnp.float32),
        pltpu.VMEM((2, _DIM, _C), jnp.float32),
        pltpu.VMEM((_DIM, _TAIL), jnp.float32),
        pltpu.SemaphoreType.DMA,
        pltpu.SemaphoreType.DMA,
        pltpu.SemaphoreType.DMA((2,)),
        pltpu.SemaphoreType.DMA((2,)),
        pltpu.SemaphoreType.DMA,
    ],
)
def _sc_fused(bank_t, si_hbm, ord_hbm, p_hbm, data_rm, out_t,
              p_v, si_v, ord_v, stage, buf, tailbuf,
              pre_sem, g_sem, in_sems, out_sems, t_sem):
    wid = lax.axis_index("s") * 2 + lax.axis_index("c")
    lo = (wid * _NWIN) // _NW
    hi = ((wid + 1) * _NWIN) // _NW
    is_last = wid == _NW - 1

    lop8 = pl.multiple_of((lo // 8) * 8, 8)
    pltpu.async_copy(p_hbm.at[pl.ds(lop8, 192)], p_v, pre_sem).wait()

    def _pread(i):
        return p_v[pl.ds(i - lop8, 16)][0]

    a = _pread(lo)
    b = _pread(jnp.where(is_last, _NWIN + 1, hi))
    n = jnp.minimum(b - a, _CAP)
    a8 = pl.multiple_of((a // 8) * 8, 8)
    off = a - a8
    pltpu.async_copy(si_hbm.at[pl.ds(a8, _CAP + 32)], si_v, pre_sem).wait()
    pltpu.async_copy(ord_hbm.at[pl.ds(a8, _CAP + 32)], ord_v, pre_sem).wait()

    nw = hi - lo

    def win_start(w):
        return pl.multiple_of(w * _C, 128)

    @pl.when(nw > 0)
    def _():
        pltpu.async_copy(
            bank_t.at[:, pl.ds(win_start(lo), _C)], buf.at[0], in_sems.at[0]
        )

    @pl.when(nw > 1)
    def _():
        pltpu.async_copy(
            bank_t.at[:, pl.ds(win_start(lo + 1), _C)], buf.at[1],
            in_sems.at[1],
        )

    def fire_stage(t, carry):
        s = ord_v[pl.ds(off + t, 16)][0]
        pltpu.async_copy(data_rm.at[s], stage.at[t], g_sem)
        return carry

    lax.fori_loop(0, n, fire_stage, 0)

    def drain_stage(t, carry):
        pltpu.make_async_copy(data_rm.at[0], stage.at[0], g_sem).wait()
        return carry

    lax.fori_loop(0, n, drain_stage, 0)

    iota16 = lax.iota(jnp.int32, 16)

    def make_apply(target):
        def apply_updates(w, wstart):
            pv = p_v[pl.ds(w - lop8, 16)]
            p0, p1 = pv[0], pv[1]

            def one(p, carry):
                c = si_v[pl.ds(off + (p - a), 16)][0] - wstart
                j = p - a
                seg = pl.multiple_of((c // 16) * 16, 16)
                mask = iota16 == c - seg
                for g in range(4):
                    xvec = stage[j, pl.ds(g * 16, 16)]
                    for l in range(16):
                        r = g * 16 + l
                        xv = jnp.broadcast_to(xvec[l], (16,))
                        cur = target(r, seg)
                        target(r, seg, jnp.where(mask, xv, cur))
                return carry

            lax.fori_loop(p0, p1, one, 0)

        return apply_updates

    def issue_in(w, slot):
        pltpu.async_copy(
            bank_t.at[:, pl.ds(win_start(w), _C)], buf.at[slot],
            in_sems.at[slot],
        )

    def wait_in(slot):
        pltpu.make_async_copy(
            bank_t.at[:, pl.ds(0, _C)], buf.at[slot], in_sems.at[slot]
        ).wait()

    def issue_out(w, slot):
        pltpu.async_copy(
            buf.at[slot], out_t.at[:, pl.ds(win_start(w), _C)],
            out_sems.at[slot],
        )

    def wait_out(slot):
        pltpu.make_async_copy(
            buf.at[slot], out_t.at[:, pl.ds(0, _C)], out_sems.at[slot]
        ).wait()

    @pl.when(nw > 0)
    def _():
        def body(i, carry):
            for slot in range(2):
                w = lo + 2 * i + slot
                nslot = 1 - slot

                def bslot_rw(r, seg, val=None):
                    if val is None:
                        return buf[slot, r, pl.ds(seg, 16)]
                    buf[slot, r, pl.ds(seg, 16)] = val

                @pl.when(w < hi)
                def _():
                    @pl.when((w + 1 < hi) & (w > lo))
                    def _():
                        wait_out(nslot)
                        issue_in(w + 1, nslot)

                    wait_in(slot)
                    make_apply(bslot_rw)(w, w * _C)
                    issue_out(w, slot)

            return carry

        lax.fori_loop(0, (nw + 1) // 2, body, 0)
        wait_out(0)

        @pl.when(nw > 1)
        def _():
            wait_out(1)

    @pl.when(is_last)
    def _():
        pltpu.async_copy(
            bank_t.at[:, pl.ds(_TAIL_START, _TAIL)], tailbuf, t_sem
        ).wait()

        def tail_rw(r, seg, val=None):
            if val is None:
                return tailbuf[r, pl.ds(seg, 16)]
            tailbuf[r, pl.ds(seg, 16)] = val

        make_apply(tail_rw)(_NWIN, _TAIL_START)
        pltpu.async_copy(
            tailbuf, out_t.at[:, pl.ds(_TAIL_START, _TAIL)], t_sem
        ).wait()


def kernel(bank, indices, data_memory):
    order = jnp.argsort(indices, stable=True).astype(jnp.int32)
    si = jnp.take(indices, order)
    wstarts = jnp.arange(_NWIN + 1, dtype=jnp.int32) * _C
    p = jnp.searchsorted(si, wstarts, side="left").astype(jnp.int32)
    p = jnp.concatenate([p, jnp.full((1,), _BATCH, jnp.int32)])
    p = jnp.pad(p, (0, _NP_PAD - _NP), constant_values=_BATCH)
    si_p = jnp.pad(si, (0, _IDX_PAD - _BATCH), constant_values=_SIZE)
    ord_p = jnp.pad(order, (0, _IDX_PAD - _BATCH))

    out_t = _sc_fused(bank.T, si_p, ord_p, p, data_memory)
    return out_t.T
